# baseline (speedup 1.0000x reference)
import functools

import jax
import jax.numpy as jnp
import numpy as np
from jax import lax
from jax.experimental import pallas as pl
from jax.experimental.pallas import tpu as pltpu
from jax.experimental.pallas import tpu_sc as plsc

N = 10000
E = 160000
F = 128
F3 = 3 * F

_NC = 2
_NS = 16
_H0 = 79360
_H1 = E - _H0
_CH = 40
_CHS = 80

_TC_PARAMS = pltpu.CompilerParams(dimension_semantics=("parallel",))


_PHI_BLK = 1000


def _pack2(a, b):
    ua = jax.lax.bitcast_convert_type(a, jnp.uint32) + jnp.uint32(0x8000)
    ub = jax.lax.bitcast_convert_type(b, jnp.uint32) + jnp.uint32(0x8000)
    return (ua >> 16) | (ub & jnp.uint32(0xFFFF0000))


def _unpack2(w):
    a = jax.lax.bitcast_convert_type(w << 16, jnp.float32)
    b = jax.lax.bitcast_convert_type(w & jnp.uint32(0xFFFF0000), jnp.float32)
    return a, b


def _phi_body(s_ref, vec_ref, w1_ref, b1_ref, w2_ref, b2_ref,
              t0_ref, t1_ref, t2_ref):
    h = jnp.dot(s_ref[...], w1_ref[...], preferred_element_type=jnp.float32)
    h = jax.nn.silu(h + b1_ref[...])
    phi = jnp.dot(h, w2_ref[...], preferred_element_type=jnp.float32) + b2_ref[...]
    v = vec_ref[...]
    phi_mid = phi[:, F:2 * F]
    t0_ref[...] = _pack2(phi[:, 0:F], phi[:, 2 * F:3 * F])
    t1_ref[...] = _pack2(phi_mid * v[:, 0:F], phi_mid * v[:, F:2 * F])
    t2_ref[...] = jax.lax.bitcast_convert_type(
        phi_mid * v[:, 2 * F:3 * F], jnp.uint32)


def _phi_tc(s, vec2, W1, b1, W2, b2):
    t = jax.ShapeDtypeStruct((N, F), jnp.uint32)
    return pl.pallas_call(
        _phi_body,
        grid=(N // _PHI_BLK,),
        in_specs=[
            pl.BlockSpec((_PHI_BLK, F), lambda i: (i, 0)),
            pl.BlockSpec((_PHI_BLK, F3), lambda i: (i, 0)),
            pl.BlockSpec((F, F), lambda i: (0, 0)),
            pl.BlockSpec((1, F), lambda i: (0, 0)),
            pl.BlockSpec((F, F3), lambda i: (0, 0)),
            pl.BlockSpec((1, F3), lambda i: (0, 0)),
        ],
        out_specs=[
            pl.BlockSpec((_PHI_BLK, F), lambda i: (i, 0)),
            pl.BlockSpec((_PHI_BLK, F), lambda i: (i, 0)),
            pl.BlockSpec((_PHI_BLK, F), lambda i: (i, 0)),
        ],
        out_shape=[t, t, t],
        compiler_params=_TC_PARAMS,
    )(s, vec2, W1, b1.reshape(1, F), W2, b2.reshape(1, F3))


def _gather_stream(tbl_hbm, out_hbm, idx_v, bufs, gsems, csems, base, nch, ch):
    def chunk_start(i, b):
        return pltpu.async_copy(
            tbl_hbm.at[idx_v.at[pl.ds(i * ch, ch)]], bufs[b], gsems[b])

    def chunk_out(i, b):
        pltpu.async_copy(bufs[b], out_hbm.at[pl.ds(base + i * ch, ch)],
                         csems[b])

    def drain_out(i, b):
        pltpu.make_async_copy(
            bufs[b], out_hbm.at[pl.ds(base + i * ch, ch)], csems[b]).wait()

    @pl.loop(0, nch // 2)
    def _(j):
        i0 = 2 * j
        for b in (0, 1):
            @pl.when(j > 0)
            def _():
                drain_out(i0 + b - 2, b)
            chunk_start(i0 + b, b)
        for b in (0, 1):
            pltpu.make_async_copy(
                tbl_hbm.at[idx_v.at[pl.ds((i0 + b) * ch, ch)]],
                bufs[b], gsems[b]).wait()
            chunk_out(i0 + b, b)

    if nch % 2:
        tail = nch - 1
        drain_out(tail - 2, 0)
        chunk_start(tail, 0).wait()
        chunk_out(tail, 0)
        drain_out(tail, 0)
        drain_out(tail - 1, 1)
    else:
        drain_out(nch - 2, 0)
        drain_out(nch - 1, 1)


def _gather_body(hbase, H, t0_hbm, t1_hbm, t2_hbm, dst_hbm,
                 g0_hbm, g1_hbm, g2_hbm,
                 idx_a, idx_b, a0, a1, c0, c1,
                 gsem0, gsem1, csem0, csem1):
    c = lax.axis_index("c")
    sid = lax.axis_index("s")
    et = H // _NS
    eth = H // (2 * _NS)
    base_a = sid * et
    pltpu.sync_copy(dst_hbm.at[pl.ds(hbase + base_a, et)], idx_a)

    @pl.when(c == 0)
    def _():
        _gather_stream(t0_hbm, g0_hbm, idx_a, (a0, a1),
                       (gsem0, gsem1), (csem0, csem1), base_a, et // _CH, _CH)
        base_b = sid * eth
        pltpu.sync_copy(dst_hbm.at[pl.ds(hbase + base_b, eth)], idx_b)
        _gather_stream(t2_hbm, g2_hbm, idx_b, (c0, c1),
                       (gsem0, gsem1), (csem0, csem1), base_b, eth // _CH, _CH)

    @pl.when(c == 1)
    def _():
        _gather_stream(t1_hbm, g1_hbm, idx_a, (a0, a1),
                       (gsem0, gsem1), (csem0, csem1), base_a, et // _CH, _CH)
        base_b = H // 2 + sid * eth
        pltpu.sync_copy(dst_hbm.at[pl.ds(hbase + base_b, eth)], idx_b)
        _gather_stream(t2_hbm, g2_hbm, idx_b, (c0, c1),
                       (gsem0, gsem1), (csem0, csem1), base_b, eth // _CH, _CH)


def _gather_sc(t0, t1, t2, dst, hbase, H):
    row = jax.ShapeDtypeStruct((H, F), jnp.uint32)
    k = pl.kernel(
        functools.partial(_gather_body, hbase, H),
        out_type=[row, row, row],
        mesh=plsc.VectorSubcoreMesh(core_axis_name="c", subcore_axis_name="s"),
        scratch_types=[
            pltpu.VMEM((H // _NS,), jnp.int32),
            pltpu.VMEM((H // (2 * _NS),), jnp.int32),
            pltpu.VMEM((_CH, F), jnp.uint32),
            pltpu.VMEM((_CH, F), jnp.uint32),
            pltpu.VMEM((_CH, F), jnp.uint32),
            pltpu.VMEM((_CH, F), jnp.uint32),
            pltpu.SemaphoreType.DMA,
            pltpu.SemaphoreType.DMA,
            pltpu.SemaphoreType.DMA,
            pltpu.SemaphoreType.DMA,
        ],
    )
    return k(t0, t1, t2, dst)


_EDGE_BLK = 1280


def _edge_body(cut_ref, g0_ref, g1_ref, g2_ref, rbf_ref, geom_ref, wr_ref,
               o_ref):
    cut = cut_ref[0, 0]
    d = geom_ref[:, 0:1]
    fcut = 0.5 * (jnp.cos(np.pi * d / cut) + 1.0)
    fcut = jnp.where(d < cut, fcut, 0.0)
    rbf17 = jnp.concatenate([rbf_ref[...] * fcut, fcut], axis=1)
    w = jnp.dot(rbf17, wr_ref[...], preferred_element_type=jnp.float32)
    phi_lo, phi_hi = _unpack2(g0_ref[...])
    p0, p1 = _unpack2(g1_ref[...])
    p2 = jax.lax.bitcast_convert_type(g2_ref[...], jnp.float32)
    w_mid = w[:, F:2 * F]
    xd = phi_hi * w[:, 2 * F:3 * F] / d
    parts = [phi_lo * w[:, 0:F]]
    for k, pk in enumerate((p0, p1, p2)):
        parts.append(pk * w_mid + geom_ref[:, k + 1:k + 2] * xd)
    o_ref[...] = jnp.concatenate(parts, axis=1)


def _edge_tc(cut_arr, g0, g1, g2, edge_rbf, geom, Wr17, blk_off, H):
    n_rbf = edge_rbf.shape[1]
    return pl.pallas_call(
        _edge_body,
        grid=(H // _EDGE_BLK,),
        in_specs=[
            pl.BlockSpec((1, 1), lambda i: (0, 0)),
            pl.BlockSpec((_EDGE_BLK, F), lambda i: (i, 0)),
            pl.BlockSpec((_EDGE_BLK, F), lambda i: (i, 0)),
            pl.BlockSpec((_EDGE_BLK, F), lambda i: (i, 0)),
            pl.BlockSpec((_EDGE_BLK, n_rbf), lambda i: (i + blk_off, 0)),
            pl.BlockSpec((_EDGE_BLK, 4), lambda i: (i + blk_off, 0)),
            pl.BlockSpec((n_rbf + 1, F3), lambda i: (0, 0)),
        ],
        out_specs=pl.BlockSpec((_EDGE_BLK, 4 * F), lambda i: (i, 0)),
        out_shape=jax.ShapeDtypeStruct((H, 4 * F), jnp.float32),
        compiler_params=_TC_PARAMS,
    )(cut_arr, g0, g1, g2, edge_rbf, geom, Wr17)


def _scatter_stream(pay_hbm, accum, idx2, bufs, lsems, ssems, blk, nch):
    col = pl.ds(blk * F, F)
    sid = lax.axis_index("s")
    base = sid * (nch * _CHS)

    def load_start(i, b):
        return pltpu.async_copy(
            pay_hbm.at[pl.ds(base + i * _CHS, _CHS), col], bufs[b], lsems[b])

    def scat_start(i, b):
        pltpu.async_copy(bufs[b], accum.at[idx2.at[i]], ssems[b], add=True)

    def drain_scat(i, b):
        pltpu.make_async_copy(
            bufs[b], accum.at[idx2.at[i]], ssems[b]).wait()

    @pl.loop(0, nch // 2)
    def _(j):
        i0 = 2 * j
        for b in (0, 1):
            @pl.when(j > 0)
            def _():
                drain_scat(i0 + b - 2, b)
            load_start(i0 + b, b)
        for b in (0, 1):
            pltpu.make_async_copy(
                pay_hbm.at[pl.ds(base + (i0 + b) * _CHS, _CHS), col],
                bufs[b], lsems[b]).wait()
            scat_start(i0 + b, b)

    if nch % 2:
        tail = nch - 1
        drain_scat(tail - 2, 0)
        load_start(tail, 0).wait()
        scat_start(tail, 0)
        drain_scat(tail, 0)
        drain_scat(tail - 1, 1)
    else:
        drain_scat(nch - 2, 0)
        drain_scat(nch - 1, 1)


def _scatter_body(nch, pay_hbm, src3_hbm, ids_hbm, idvec_hbm, ds_hbm, dvec_hbm,
                  idx2, b0, b1, accum, lsem0, lsem1, ssem0, ssem1):
    c = lax.axis_index("c")
    sid = lax.axis_index("s")
    pltpu.sync_copy(src3_hbm.at[sid], idx2)

    for blk in range(4):
        @pl.when(c == (blk % 2))
        def _():
            plsc.subcore_barrier()

            @pl.when(sid == 0)
            def _():
                if blk == 0:
                    pltpu.sync_copy(ids_hbm, accum)
                else:
                    pltpu.sync_copy(idvec_hbm.at[blk - 1], accum)

            plsc.subcore_barrier()
            _scatter_stream(pay_hbm, accum, idx2, (b0, b1),
                            (lsem0, lsem1), (ssem0, ssem1), blk, nch)
            plsc.subcore_barrier()

            @pl.when(sid == 0)
            def _():
                if blk == 0:
                    pltpu.sync_copy(accum, ds_hbm)
                else:
                    pltpu.sync_copy(accum, dvec_hbm.at[blk - 1])


def _scatter_sc(pay, src3, init_ds, init_dvec, nch):
    k = pl.kernel(
        functools.partial(_scatter_body, nch),
        out_type=[
            jax.ShapeDtypeStruct((N, F), jnp.float32),
            jax.ShapeDtypeStruct((3, N, F), jnp.float32),
        ],
        mesh=plsc.VectorSubcoreMesh(core_axis_name="c", subcore_axis_name="s"),
        scratch_types=[
            pltpu.VMEM((nch, _CHS), jnp.int32),
            pltpu.VMEM((_CHS, F), jnp.float32),
            pltpu.VMEM((_CHS, F), jnp.float32),
            pltpu.VMEM_SHARED((N, F), jnp.float32),
            pltpu.SemaphoreType.DMA,
            pltpu.SemaphoreType.DMA,
            pltpu.SemaphoreType.DMA,
            pltpu.SemaphoreType.DMA,
        ],
    )
    return k(pay, src3, init_ds, init_dvec)


def kernel(s, vec, edge_indexes, edge_vector, edge_distance, edge_rbf,
           cutoff_dist, W1, b1, W2, b2, Wr, br):
    src = edge_indexes[0].astype(jnp.int32)
    dst = edge_indexes[1].astype(jnp.int32)
    vec2 = vec.reshape(N, F3)
    cut_arr = jnp.asarray(cutoff_dist, jnp.float32).reshape(1, 1)
    geom = jnp.concatenate(
        [edge_distance.reshape(E, 1), edge_vector], axis=1)
    zeros_ds = jnp.zeros((N, F), jnp.float32)
    zeros_dvec = jnp.zeros((3, N, F), jnp.float32)
    Wr17 = jnp.concatenate([Wr, br.reshape(1, F3)], axis=0)

    t0, t1, t2 = _phi_tc(s, vec2, W1, b1, W2, b2)

    chunks = ((0, 39680), (39680, 39680), (79360, 39680), (119040, 40960))
    pays = []
    for hbase, H in chunks:
        g0, g1, g2 = _gather_sc(t0, t1, t2, dst, hbase, H)
        pays.append(
            _edge_tc(cut_arr, g0, g1, g2, edge_rbf, geom, Wr17,
                     hbase // _EDGE_BLK, H))

    ds, dvec = zeros_ds, zeros_dvec
    for pay, (hbase, H) in zip(pays, chunks):
        nch = H // _NS // _CHS
        src3 = src[hbase:hbase + H].reshape(_NS, nch, _CHS)
        ds, dvec = _scatter_sc(pay, src3, ds, dvec, nch)
    return ds, dvec.transpose(1, 0, 2)


# speedup vs baseline: 1.0872x; 1.0872x over previous
import functools

import jax
import jax.numpy as jnp
import numpy as np
from jax import lax
from jax.experimental import pallas as pl
from jax.experimental.pallas import tpu as pltpu
from jax.experimental.pallas import tpu_sc as plsc

N = 10000
E = 160000
F = 128
F3 = 3 * F

_NC = 2
_NS = 16
_H0 = 79360
_H1 = E - _H0
_CHA = 80
_CHB = 40
_CHS = 80

_TC_PARAMS = pltpu.CompilerParams(dimension_semantics=("parallel",))


_PHI_BLK = 1000


def _pack2(a, b):
    ua = jax.lax.bitcast_convert_type(a, jnp.uint32) + jnp.uint32(0x8000)
    ub = jax.lax.bitcast_convert_type(b, jnp.uint32) + jnp.uint32(0x8000)
    return (ua >> 16) | (ub & jnp.uint32(0xFFFF0000))


def _unpack2(w):
    a = jax.lax.bitcast_convert_type(w << 16, jnp.float32)
    b = jax.lax.bitcast_convert_type(w & jnp.uint32(0xFFFF0000), jnp.float32)
    return a, b


def _phi_body(s_ref, vec_ref, w1_ref, b1_ref, w2_ref, b2_ref,
              t0_ref, t1_ref, t2_ref):
    h = jnp.dot(s_ref[...], w1_ref[...], preferred_element_type=jnp.float32)
    h = jax.nn.silu(h + b1_ref[...])
    phi = jnp.dot(h, w2_ref[...], preferred_element_type=jnp.float32) + b2_ref[...]
    v = vec_ref[...]
    phi_mid = phi[:, F:2 * F]
    t0_ref[...] = _pack2(phi[:, 0:F], phi[:, 2 * F:3 * F])
    t1_ref[...] = _pack2(phi_mid * v[:, 0:F], phi_mid * v[:, F:2 * F])
    t2_ref[...] = jax.lax.bitcast_convert_type(
        phi_mid * v[:, 2 * F:3 * F], jnp.uint32)


def _phi_tc(s, vec2, W1, b1, W2, b2):
    t = jax.ShapeDtypeStruct((N, F), jnp.uint32)
    return pl.pallas_call(
        _phi_body,
        grid=(N // _PHI_BLK,),
        in_specs=[
            pl.BlockSpec((_PHI_BLK, F), lambda i: (i, 0)),
            pl.BlockSpec((_PHI_BLK, F3), lambda i: (i, 0)),
            pl.BlockSpec((F, F), lambda i: (0, 0)),
            pl.BlockSpec((1, F), lambda i: (0, 0)),
            pl.BlockSpec((F, F3), lambda i: (0, 0)),
            pl.BlockSpec((1, F3), lambda i: (0, 0)),
        ],
        out_specs=[
            pl.BlockSpec((_PHI_BLK, F), lambda i: (i, 0)),
            pl.BlockSpec((_PHI_BLK, F), lambda i: (i, 0)),
            pl.BlockSpec((_PHI_BLK, F), lambda i: (i, 0)),
        ],
        out_shape=[t, t, t],
        compiler_params=_TC_PARAMS,
    )(s, vec2, W1, b1.reshape(1, F), W2, b2.reshape(1, F3))


def _gather_stream(tbl_hbm, out_hbm, idx_v, bufs, gsems, csems, base, nch, ch):
    def chunk_start(i, b):
        return pltpu.async_copy(
            tbl_hbm.at[idx_v.at[pl.ds(i * ch, ch)]], bufs[b], gsems[b])

    def chunk_out(i, b):
        pltpu.async_copy(bufs[b], out_hbm.at[pl.ds(base + i * ch, ch)],
                         csems[b])

    def drain_out(i, b):
        pltpu.make_async_copy(
            bufs[b], out_hbm.at[pl.ds(base + i * ch, ch)], csems[b]).wait()

    @pl.loop(0, nch // 2)
    def _(j):
        i0 = 2 * j
        for b in (0, 1):
            @pl.when(j > 0)
            def _():
                drain_out(i0 + b - 2, b)
            chunk_start(i0 + b, b)
        for b in (0, 1):
            pltpu.make_async_copy(
                tbl_hbm.at[idx_v.at[pl.ds((i0 + b) * ch, ch)]],
                bufs[b], gsems[b]).wait()
            chunk_out(i0 + b, b)

    if nch % 2:
        tail = nch - 1
        drain_out(tail - 2, 0)
        chunk_start(tail, 0).wait()
        chunk_out(tail, 0)
        drain_out(tail, 0)
        drain_out(tail - 1, 1)
    else:
        drain_out(nch - 2, 0)
        drain_out(nch - 1, 1)


def _gather_body(hbase, H, t0_hbm, t1_hbm, t2_hbm, dst_hbm,
                 g0_hbm, g1_hbm, g2_hbm,
                 idx_a, idx_b, a0, a1, c0, c1,
                 gsem0, gsem1, csem0, csem1):
    c = lax.axis_index("c")
    sid = lax.axis_index("s")
    et = H // _NS
    eth = H // (2 * _NS)
    base_a = sid * et
    pltpu.sync_copy(dst_hbm.at[pl.ds(hbase + base_a, et)], idx_a)

    @pl.when(c == 0)
    def _():
        _gather_stream(t0_hbm, g0_hbm, idx_a, (a0, a1),
                       (gsem0, gsem1), (csem0, csem1), base_a, et // _CHA, _CHA)
        base_b = sid * eth
        pltpu.sync_copy(dst_hbm.at[pl.ds(hbase + base_b, eth)], idx_b)
        _gather_stream(t2_hbm, g2_hbm, idx_b, (c0, c1),
                       (gsem0, gsem1), (csem0, csem1), base_b, eth // _CHB, _CHB)

    @pl.when(c == 1)
    def _():
        _gather_stream(t1_hbm, g1_hbm, idx_a, (a0, a1),
                       (gsem0, gsem1), (csem0, csem1), base_a, et // _CHA, _CHA)
        base_b = H // 2 + sid * eth
        pltpu.sync_copy(dst_hbm.at[pl.ds(hbase + base_b, eth)], idx_b)
        _gather_stream(t2_hbm, g2_hbm, idx_b, (c0, c1),
                       (gsem0, gsem1), (csem0, csem1), base_b, eth // _CHB, _CHB)


def _gather_sc(t0, t1, t2, dst, hbase, H):
    row = jax.ShapeDtypeStruct((H, F), jnp.uint32)
    k = pl.kernel(
        functools.partial(_gather_body, hbase, H),
        out_type=[row, row, row],
        mesh=plsc.VectorSubcoreMesh(core_axis_name="c", subcore_axis_name="s"),
        scratch_types=[
            pltpu.VMEM((H // _NS,), jnp.int32),
            pltpu.VMEM((H // (2 * _NS),), jnp.int32),
            pltpu.VMEM((_CHA, F), jnp.uint32),
            pltpu.VMEM((_CHA, F), jnp.uint32),
            pltpu.VMEM((_CHB, F), jnp.uint32),
            pltpu.VMEM((_CHB, F), jnp.uint32),
            pltpu.SemaphoreType.DMA,
            pltpu.SemaphoreType.DMA,
            pltpu.SemaphoreType.DMA,
            pltpu.SemaphoreType.DMA,
        ],
    )
    return k(t0, t1, t2, dst)


_EDGE_BLK = 1280


def _edge_body(cut_ref, g0_ref, g1_ref, g2_ref, rbf_ref, geom_ref, wr_ref,
               o_ref):
    cut = cut_ref[0, 0]
    d = geom_ref[:, 0:1]
    fcut = 0.5 * (jnp.cos(np.pi * d / cut) + 1.0)
    fcut = jnp.where(d < cut, fcut, 0.0)
    rbf17 = jnp.concatenate([rbf_ref[...] * fcut, fcut], axis=1)
    w = jnp.dot(rbf17, wr_ref[...], preferred_element_type=jnp.float32)
    phi_lo, phi_hi = _unpack2(g0_ref[...])
    p0, p1 = _unpack2(g1_ref[...])
    p2 = jax.lax.bitcast_convert_type(g2_ref[...], jnp.float32)
    w_mid = w[:, F:2 * F]
    xd = phi_hi * w[:, 2 * F:3 * F] / d
    parts = [phi_lo * w[:, 0:F]]
    for k, pk in enumerate((p0, p1, p2)):
        parts.append(pk * w_mid + geom_ref[:, k + 1:k + 2] * xd)
    o_ref[...] = jnp.concatenate(parts, axis=1)


def _edge_tc(cut_arr, g0, g1, g2, edge_rbf, geom, Wr17, blk_off, H):
    n_rbf = edge_rbf.shape[1]
    return pl.pallas_call(
        _edge_body,
        grid=(H // _EDGE_BLK,),
        in_specs=[
            pl.BlockSpec((1, 1), lambda i: (0, 0)),
            pl.BlockSpec((_EDGE_BLK, F), lambda i: (i, 0)),
            pl.BlockSpec((_EDGE_BLK, F), lambda i: (i, 0)),
            pl.BlockSpec((_EDGE_BLK, F), lambda i: (i, 0)),
            pl.BlockSpec((_EDGE_BLK, n_rbf), lambda i: (i + blk_off, 0)),
            pl.BlockSpec((_EDGE_BLK, 4), lambda i: (i + blk_off, 0)),
            pl.BlockSpec((n_rbf + 1, F3), lambda i: (0, 0)),
        ],
        out_specs=pl.BlockSpec((_EDGE_BLK, 4 * F), lambda i: (i, 0)),
        out_shape=jax.ShapeDtypeStruct((H, 4 * F), jnp.float32),
        compiler_params=_TC_PARAMS,
    )(cut_arr, g0, g1, g2, edge_rbf, geom, Wr17)


def _scatter_stream(pay_hbm, accum, idx2, bufs, lsems, ssems, blk, nch):
    col = pl.ds(blk * F, F)
    sid = lax.axis_index("s")
    base = sid * (nch * _CHS)

    def load_start(i, b):
        return pltpu.async_copy(
            pay_hbm.at[pl.ds(base + i * _CHS, _CHS), col], bufs[b], lsems[b])

    def scat_start(i, b):
        pltpu.async_copy(bufs[b], accum.at[idx2.at[i]], ssems[b], add=True)

    def drain_scat(i, b):
        pltpu.make_async_copy(
            bufs[b], accum.at[idx2.at[i]], ssems[b]).wait()

    @pl.loop(0, nch // 2)
    def _(j):
        i0 = 2 * j
        for b in (0, 1):
            @pl.when(j > 0)
            def _():
                drain_scat(i0 + b - 2, b)
            load_start(i0 + b, b)
        for b in (0, 1):
            pltpu.make_async_copy(
                pay_hbm.at[pl.ds(base + (i0 + b) * _CHS, _CHS), col],
                bufs[b], lsems[b]).wait()
            scat_start(i0 + b, b)

    if nch % 2:
        tail = nch - 1
        drain_scat(tail - 2, 0)
        load_start(tail, 0).wait()
        scat_start(tail, 0)
        drain_scat(tail, 0)
        drain_scat(tail - 1, 1)
    else:
        drain_scat(nch - 2, 0)
        drain_scat(nch - 1, 1)


def _scatter_body(nch, pay_hbm, src3_hbm, ids_hbm, idvec_hbm, ds_hbm, dvec_hbm,
                  idx2, b0, b1, accum, lsem0, lsem1, ssem0, ssem1):
    c = lax.axis_index("c")
    sid = lax.axis_index("s")
    pltpu.sync_copy(src3_hbm.at[sid], idx2)

    for blk in range(4):
        @pl.when(c == (blk % 2))
        def _():
            plsc.subcore_barrier()

            @pl.when(sid == 0)
            def _():
                if blk == 0:
                    pltpu.sync_copy(ids_hbm, accum)
                else:
                    pltpu.sync_copy(idvec_hbm.at[blk - 1], accum)

            plsc.subcore_barrier()
            _scatter_stream(pay_hbm, accum, idx2, (b0, b1),
                            (lsem0, lsem1), (ssem0, ssem1), blk, nch)
            plsc.subcore_barrier()

            @pl.when(sid == 0)
            def _():
                if blk == 0:
                    pltpu.sync_copy(accum, ds_hbm)
                else:
                    pltpu.sync_copy(accum, dvec_hbm.at[blk - 1])


def _scatter_sc(pay, src3, init_ds, init_dvec, nch):
    k = pl.kernel(
        functools.partial(_scatter_body, nch),
        out_type=[
            jax.ShapeDtypeStruct((N, F), jnp.float32),
            jax.ShapeDtypeStruct((3, N, F), jnp.float32),
        ],
        mesh=plsc.VectorSubcoreMesh(core_axis_name="c", subcore_axis_name="s"),
        scratch_types=[
            pltpu.VMEM((nch, _CHS), jnp.int32),
            pltpu.VMEM((_CHS, F), jnp.float32),
            pltpu.VMEM((_CHS, F), jnp.float32),
            pltpu.VMEM_SHARED((N, F), jnp.float32),
            pltpu.SemaphoreType.DMA,
            pltpu.SemaphoreType.DMA,
            pltpu.SemaphoreType.DMA,
            pltpu.SemaphoreType.DMA,
        ],
    )
    return k(pay, src3, init_ds, init_dvec)


def kernel(s, vec, edge_indexes, edge_vector, edge_distance, edge_rbf,
           cutoff_dist, W1, b1, W2, b2, Wr, br):
    src = edge_indexes[0].astype(jnp.int32)
    dst = edge_indexes[1].astype(jnp.int32)
    vec2 = vec.reshape(N, F3)
    cut_arr = jnp.asarray(cutoff_dist, jnp.float32).reshape(1, 1)
    geom = jnp.concatenate(
        [edge_distance.reshape(E, 1), edge_vector], axis=1)
    zeros_ds = jnp.zeros((N, F), jnp.float32)
    zeros_dvec = jnp.zeros((3, N, F), jnp.float32)
    Wr17 = jnp.concatenate([Wr, br.reshape(1, F3)], axis=0)

    t0, t1, t2 = _phi_tc(s, vec2, W1, b1, W2, b2)

    chunks = ((0, _H0), (_H0, _H1))
    pays = []
    for hbase, H in chunks:
        g0, g1, g2 = _gather_sc(t0, t1, t2, dst, hbase, H)
        pays.append(
            _edge_tc(cut_arr, g0, g1, g2, edge_rbf, geom, Wr17,
                     hbase // _EDGE_BLK, H))

    ds, dvec = zeros_ds, zeros_dvec
    for pay, (hbase, H) in zip(pays, chunks):
        nch = H // _NS // _CHS
        src3 = src[hbase:hbase + H].reshape(_NS, nch, _CHS)
        ds, dvec = _scatter_sc(pay, src3, ds, dvec, nch)
    return ds, dvec.transpose(1, 0, 2)
